# baseline (speedup 1.0000x reference)
import jax
import jax.numpy as jnp
from jax.experimental import pallas as pl
from jax.experimental.pallas import tpu as pltpu

_ROWS = 5000
_N_CHUNKS = 4
_CH = _ROWS // _N_CHUNKS


def _dma_pipe_kernel(src, dst, buf0, buf1, in_sems, out_sems):
    bufs = (buf0, buf1)

    def in_copy(i):
        return pltpu.make_async_copy(
            src.at[pl.ds(i * _CH, _CH)], bufs[i % 2], in_sems.at[i])

    def out_copy(i):
        return pltpu.make_async_copy(
            bufs[i % 2], dst.at[pl.ds(i * _CH, _CH)], out_sems.at[i])

    in_copy(0).start()
    for i in range(_N_CHUNKS):
        in_copy(i).wait()
        out_copy(i).start()
        if i + 1 < _N_CHUNKS:
            if i >= 1:
                out_copy(i - 1).wait()
            in_copy(i + 1).start()
    out_copy(_N_CHUNKS - 2).wait()
    out_copy(_N_CHUNKS - 1).wait()


def kernel(node_features, edges, layer_lengths, W1, b1, W2, b2):
    n = edges.shape[0] * edges.shape[1]
    flat = edges.reshape(_ROWS, 128)
    out = pl.pallas_call(
        _dma_pipe_kernel,
        in_specs=[pl.BlockSpec(memory_space=pl.ANY)],
        out_specs=pl.BlockSpec(memory_space=pl.ANY),
        out_shape=jax.ShapeDtypeStruct(flat.shape, flat.dtype),
        scratch_shapes=[
            pltpu.VMEM((_CH, 128), jnp.int32),
            pltpu.VMEM((_CH, 128), jnp.int32),
            pltpu.SemaphoreType.DMA((_N_CHUNKS,)),
            pltpu.SemaphoreType.DMA((_N_CHUNKS,)),
        ],
    )(flat)
    return out.reshape(edges.shape)


# speedup vs baseline: 1.2769x; 1.2769x over previous
import jax
import jax.numpy as jnp
from jax.experimental import pallas as pl
from jax.experimental.pallas import tpu as pltpu

_ROWS = 5000
_N_CHUNKS = 4
_CH = _ROWS // _N_CHUNKS


def _dma_pipe_kernel(src, dst, buf0, buf1, buf2, buf3, in_sems, out_sems):
    bufs = (buf0, buf1, buf2, buf3)

    def in_copy(i):
        return pltpu.make_async_copy(
            src.at[pl.ds(i * _CH, _CH)], bufs[i], in_sems.at[i])

    def out_copy(i):
        return pltpu.make_async_copy(
            bufs[i], dst.at[pl.ds(i * _CH, _CH)], out_sems.at[i])

    for i in range(_N_CHUNKS):
        in_copy(i).start()
    for i in range(_N_CHUNKS):
        in_copy(i).wait()
        out_copy(i).start()
    for i in range(_N_CHUNKS):
        out_copy(i).wait()


def kernel(node_features, edges, layer_lengths, W1, b1, W2, b2):
    flat = edges.reshape(_ROWS, 128)
    out = pl.pallas_call(
        _dma_pipe_kernel,
        in_specs=[pl.BlockSpec(memory_space=pl.ANY)],
        out_specs=pl.BlockSpec(memory_space=pl.ANY),
        out_shape=jax.ShapeDtypeStruct(flat.shape, flat.dtype),
        scratch_shapes=[
            pltpu.VMEM((_CH, 128), jnp.int32),
            pltpu.VMEM((_CH, 128), jnp.int32),
            pltpu.VMEM((_CH, 128), jnp.int32),
            pltpu.VMEM((_CH, 128), jnp.int32),
            pltpu.SemaphoreType.DMA((_N_CHUNKS,)),
            pltpu.SemaphoreType.DMA((_N_CHUNKS,)),
        ],
    )(flat)
    return out.reshape(edges.shape)
